# baseline (speedup 1.0000x reference)
import functools

import jax
import jax.numpy as jnp
from jax import lax
from jax.experimental import pallas as pl
from jax.experimental.pallas import tpu as pltpu
from jax.experimental.pallas import tpu_sc as plsc

N_NODES = 10000
N_PAD = 10240
EMB = 128
HALF = EMB // 2
N_HALVES = 2
P = N_PAD // 2
N_EDGES = 320000
N_CORES = 2
N_TILES = 16
N_WORKERS = N_CORES * N_TILES
CHUNK = 128
SUPC = 16
NSUP = 5
IDXROWS = SUPC
CHUNKS_PER_TILE = IDXROWS * NSUP
CHUNKS_PER_CORE = CHUNKS_PER_TILE * N_TILES
N_IDXROWS = CHUNKS_PER_CORE * N_CORES
EDGES_PAD = N_IDXROWS * 128
PROWS_PER_TILE = P // N_TILES
PROWS_PER_WORKER = P // N_WORKERS
ZROWS = 8
FBLK = 80

_mesh = plsc.VectorSubcoreMesh(core_axis_name="c", subcore_axis_name="s",
                               num_cores=N_CORES)


def _scale_chunk(gbuf, cidx2, ridx2, vals2, jj):
    row = jj
    one = jnp.float32(1.0)

    def grp_body(g, carry):
        sl16 = pl.ds(g * 16, 16)
        vv = vals2[row, sl16]
        cc = cidx2[row, sl16]
        rr_ = ridx2[row, sl16]
        for k in range(16):
            v = vv[k]
            pc = (cc[k] & 1).astype(jnp.float32)
            pr = (rr_[k] & 1).astype(jnp.float32)
            a = v * (one - pc)
            b = v * pc
            c = one - pr
            ac = jnp.full((16,), a * c, jnp.float32)
            bc = jnp.full((16,), b * c, jnp.float32)
            ad = jnp.full((16,), a * pr, jnp.float32)
            bd = jnp.full((16,), b * pr, jnp.float32)
            r = g * 16 + k
            for q in range(4):
                lo_sl = pl.ds(q * 16, 16)
                hi_sl = pl.ds(HALF + q * 16, 16)
                lo = gbuf[r, lo_sl]
                hi = gbuf[r, hi_sl]
                gbuf[r, lo_sl] = lo * ac + hi * bc
                gbuf[r, hi_sl] = lo * ad + hi * bd
        return carry
    lax.fori_loop(0, CHUNK // 16, grp_body, 0, unroll=False)


def _layer(cid, tid, xs_sh, rows2_hbm, cols2_hbm, vals2_hbm, acc_sh,
           cidx2, ridx2, vals2, csh2, rsh2, gb0, gb1,
           sem0, sem1, ssem0, ssem1):
    gb = (gb0, gb1)
    gsem = (sem0, sem1)
    ssem = (ssem0, ssem1)

    def super_body(j, carry):
        r0 = cid * CHUNKS_PER_CORE + tid * CHUNKS_PER_TILE + j * IDXROWS
        pltpu.sync_copy(cols2_hbm.at[pl.ds(r0, IDXROWS)], cidx2)
        pltpu.sync_copy(rows2_hbm.at[pl.ds(r0, IDXROWS)], ridx2)
        pltpu.sync_copy(vals2_hbm.at[pl.ds(r0, IDXROWS)], vals2)

        def shift_body(m, carry):
            for g in range(8):
                sl = pl.ds(g * 16, 16)
                csh2[m, sl] = lax.shift_right_logical(cidx2[m, sl], 1)
                rsh2[m, sl] = lax.shift_right_logical(ridx2[m, sl], 1)
            return carry
        lax.fori_loop(0, IDXROWS, shift_body, 0, unroll=False)

        pending = pltpu.async_copy(xs_sh.at[csh2.at[0]], gb0, sem0)
        scat = [None, None]
        for jj in range(SUPC):
            b = jj % 2
            cur = gb[b]
            if jj < SUPC - 1:
                b2 = 1 - b
                if scat[b2] is not None:
                    scat[b2].wait()
                    scat[b2] = None
                nxt_pending = pltpu.async_copy(
                    xs_sh.at[csh2.at[jj + 1]], gb[b2], gsem[b2])
            pending.wait()
            _scale_chunk(cur, cidx2, ridx2, vals2, jj)
            scat[b] = pltpu.async_copy(cur, acc_sh.at[rsh2.at[jj]], ssem[b],
                                       add=True)
            if jj < SUPC - 1:
                pending = nxt_pending
        for b in range(2):
            if scat[b] is not None:
                scat[b].wait()
        return carry

    lax.fori_loop(0, NSUP, super_body, 0, unroll=False)


@functools.partial(
    pl.kernel,
    out_type=jax.ShapeDtypeStruct((N_CORES, N_HALVES, P, EMB), jnp.float32),
    mesh=_mesh,
    scratch_types=[
        pltpu.VMEM_SHARED((P, EMB), jnp.float32),
        pltpu.VMEM_SHARED((P, EMB), jnp.float32),
        pltpu.VMEM((IDXROWS, 128), jnp.int32),
        pltpu.VMEM((IDXROWS, 128), jnp.int32),
        pltpu.VMEM((IDXROWS, 128), jnp.float32),
        pltpu.VMEM((IDXROWS, 128), jnp.int32),
        pltpu.VMEM((IDXROWS, 128), jnp.int32),
        pltpu.VMEM((CHUNK, EMB), jnp.float32),
        pltpu.VMEM((CHUNK, EMB), jnp.float32),
        pltpu.VMEM((ZROWS, EMB), jnp.float32),
        pltpu.SemaphoreType.DMA,
        pltpu.SemaphoreType.DMA,
        pltpu.SemaphoreType.DMA,
        pltpu.SemaphoreType.DMA,
    ],
)
def _spmm_partial(src_hbm, vals2_hbm, rows2_hbm, cols2_hbm, part_hbm,
                  xs_sh, acc_sh, cidx2, ridx2, vals2, csh2, rsh2,
                  gb0, gb1, zbuf, sem0, sem1, ssem0, ssem1):
    cid = lax.axis_index("c")
    tid = lax.axis_index("s")
    prow0 = tid * PROWS_PER_TILE
    zero16 = jnp.zeros((16,), jnp.float32)

    def zrow(r, carry):
        for q in range(EMB // 16):
            zbuf[r, pl.ds(q * 16, 16)] = zero16
        return carry
    lax.fori_loop(0, ZROWS, zrow, 0, unroll=False)

    for h in range(N_HALVES):
        pltpu.sync_copy(src_hbm.at[h].at[pl.ds(prow0, PROWS_PER_TILE)],
                        xs_sh.at[pl.ds(prow0, PROWS_PER_TILE)])
        for k in range(PROWS_PER_TILE // ZROWS):
            pltpu.sync_copy(zbuf, acc_sh.at[pl.ds(prow0 + k * ZROWS, ZROWS)])
        plsc.subcore_barrier()

        _layer(cid, tid, xs_sh, rows2_hbm, cols2_hbm, vals2_hbm, acc_sh,
               cidx2, ridx2, vals2, csh2, rsh2, gb0, gb1,
               sem0, sem1, ssem0, ssem1)
        plsc.subcore_barrier()

        pltpu.sync_copy(acc_sh.at[pl.ds(prow0, PROWS_PER_TILE)],
                        part_hbm.at[cid].at[h].at[pl.ds(prow0, PROWS_PER_TILE)])
        if h + 1 < N_HALVES:
            plsc.subcore_barrier()


@functools.partial(
    pl.kernel,
    out_type=jax.ShapeDtypeStruct((N_HALVES, P, EMB), jnp.float32),
    mesh=_mesh,
    scratch_types=[
        pltpu.VMEM((FBLK, EMB), jnp.float32),
        pltpu.VMEM((FBLK, EMB), jnp.float32),
    ],
)
def _combine2(part_hbm, x1_hbm, bufa, bufb):
    wid = lax.axis_index("c") * N_TILES + lax.axis_index("s")
    row0 = wid * PROWS_PER_WORKER
    for h in range(N_HALVES):
        for k in range(PROWS_PER_WORKER // FBLK):
            r = row0 + k * FBLK
            pltpu.sync_copy(part_hbm.at[0].at[h].at[pl.ds(r, FBLK)], bufa)
            pltpu.sync_copy(part_hbm.at[1].at[h].at[pl.ds(r, FBLK)], bufb)

            def frow(rr, carry):
                for q in range(EMB // 16):
                    sl = pl.ds(q * 16, 16)
                    bufa[rr, sl] = bufa[rr, sl] + bufb[rr, sl]
                return carry
            lax.fori_loop(0, FBLK, frow, 0, unroll=False)

            pltpu.sync_copy(bufa, x1_hbm.at[h].at[pl.ds(r, FBLK)])


@functools.partial(
    pl.kernel,
    out_type=jax.ShapeDtypeStruct((N_HALVES, P, EMB), jnp.float32),
    mesh=_mesh,
    scratch_types=[
        pltpu.VMEM((FBLK, EMB), jnp.float32),
        pltpu.VMEM((FBLK, EMB), jnp.float32),
    ],
)
def _final4(emb_hbm, x1_hbm, part_hbm, out_hbm, bufa, bufb):
    wid = lax.axis_index("c") * N_TILES + lax.axis_index("s")
    row0 = wid * PROWS_PER_WORKER
    third = jnp.float32(1.0 / 3.0)
    for h in range(N_HALVES):
        for k in range(PROWS_PER_WORKER // FBLK):
            r = row0 + k * FBLK
            pltpu.sync_copy(emb_hbm.at[h].at[pl.ds(r, FBLK)], bufa)
            pltpu.sync_copy(x1_hbm.at[h].at[pl.ds(r, FBLK)], bufb)

            def add_rows(rr, carry):
                for q in range(EMB // 16):
                    sl = pl.ds(q * 16, 16)
                    bufa[rr, sl] = bufa[rr, sl] + bufb[rr, sl]
                return carry

            lax.fori_loop(0, FBLK, add_rows, 0, unroll=False)
            pltpu.sync_copy(part_hbm.at[0].at[h].at[pl.ds(r, FBLK)], bufb)
            lax.fori_loop(0, FBLK, add_rows, 0, unroll=False)
            pltpu.sync_copy(part_hbm.at[1].at[h].at[pl.ds(r, FBLK)], bufb)

            def fin_rows(rr, carry):
                for q in range(EMB // 16):
                    sl = pl.ds(q * 16, 16)
                    bufa[rr, sl] = (bufa[rr, sl] + bufb[rr, sl]) * third
                return carry
            lax.fori_loop(0, FBLK, fin_rows, 0, unroll=False)

            pltpu.sync_copy(bufa, out_hbm.at[h].at[pl.ds(r, FBLK)])


def _pack(x):
    return x.reshape(P, 2, N_HALVES, HALF).transpose(2, 0, 1, 3).reshape(
        N_HALVES, P, EMB)


def _unpack(xs):
    return xs.reshape(N_HALVES, P, 2, HALF).transpose(1, 2, 0, 3).reshape(
        N_PAD, EMB)


def kernel(embedding, adj_values, edge_index):
    rows = edge_index[0].astype(jnp.int32)
    cols = edge_index[1].astype(jnp.int32)
    vals = adj_values.astype(jnp.float32)
    pad = EDGES_PAD - N_EDGES
    rows2 = jnp.pad(rows, (0, pad)).reshape(N_IDXROWS, 128)
    cols2 = jnp.pad(cols, (0, pad)).reshape(N_IDXROWS, 128)
    vals2 = jnp.pad(vals, (0, pad)).reshape(N_IDXROWS, 128)
    emb = jnp.pad(embedding.astype(jnp.float32), ((0, N_PAD - N_NODES), (0, 0)))
    emb_s = _pack(emb)
    parts1 = _spmm_partial(emb_s, vals2, rows2, cols2)
    x1_s = _combine2(parts1)
    parts2 = _spmm_partial(x1_s, vals2, rows2, cols2)
    out_s = _final4(emb_s, x1_s, parts2)
    return _unpack(out_s)[:N_NODES]
